# baseline (speedup 1.0000x reference)
import functools

import jax
import jax.numpy as jnp
from jax import lax
from jax.experimental import pallas as pl
from jax.experimental.pallas import tpu as pltpu
from jax.experimental.pallas import tpu_sc as plsc

_N = 10000
_E = 320000
_H = 128
_HH = 64

_CE = 64
_NCHUNK = _E // _CE
_NS = 16
_ITERS = -(-_NCHUNK // _NS)
_NROW = _N // _NS
_RC = 125

_EB = 512
_NB = 512


def _embed_edge_body(e_ref, we_ref, be_ref, he_ref):
    o = e_ref[...] * we_ref[...] + be_ref[...]
    he_ref[0, :, :] = o[:, :_HH]
    he_ref[1, :, :] = o[:, _HH:]


def _embed_edge(e2, we, be):
    grid = _E // _EB
    return pl.pallas_call(
        _embed_edge_body,
        grid=(grid,),
        in_specs=[
            pl.BlockSpec((_EB, 1), lambda i: (i, 0)),
            pl.BlockSpec((1, _H), lambda i: (0, 0)),
            pl.BlockSpec((1, _H), lambda i: (0, 0)),
        ],
        out_specs=pl.BlockSpec((2, _EB, _HH), lambda i: (0, i, 0)),
        out_shape=jax.ShapeDtypeStruct((2, _E, _HH), jnp.float32),
    )(e2, we, be)


def _node_tf_body(h_ref, w_ref, b_ref, ux_ref, tsa_ref, tsb_ref, tdc_ref):
    h = h_ref[...]
    o = jnp.dot(h, w_ref[...], preferred_element_type=jnp.float32) + b_ref[...]
    vx = o[:, _H:2 * _H]
    hb = o[:, 2 * _H:3 * _H]
    hc = o[:, 3 * _H:]
    ux_ref[...] = o[:, :_H]
    tsa_ref[...] = jnp.concatenate([vx[:, :_HH], hb[:, :_HH]], axis=1)
    tsb_ref[...] = jnp.concatenate([vx[:, _HH:], hb[:, _HH:]], axis=1)
    tdc_ref[...] = hc


def _node_tf0_body(x_ref, wx_ref, bx_ref, w_ref, b_ref, h0_ref, ux_ref,
                   tsa_ref, tsb_ref, tdc_ref):
    h = (jnp.dot(x_ref[...], wx_ref[...], preferred_element_type=jnp.float32)
         + bx_ref[...])
    h0_ref[...] = h
    o = jnp.dot(h, w_ref[...], preferred_element_type=jnp.float32) + b_ref[...]
    vx = o[:, _H:2 * _H]
    hb = o[:, 2 * _H:3 * _H]
    hc = o[:, 3 * _H:]
    ux_ref[...] = o[:, :_H]
    tsa_ref[...] = jnp.concatenate([vx[:, :_HH], hb[:, :_HH]], axis=1)
    tsb_ref[...] = jnp.concatenate([vx[:, _HH:], hb[:, _HH:]], axis=1)
    tdc_ref[...] = hc


_NODE_OUT_SHAPES = [
    jax.ShapeDtypeStruct((_N, _H), jnp.float32),
    jax.ShapeDtypeStruct((_N, _H), jnp.float32),
    jax.ShapeDtypeStruct((_N, _H), jnp.float32),
    jax.ShapeDtypeStruct((_N, _H), jnp.float32),
]
_NODE_OUT_SPECS = [
    pl.BlockSpec((_NB, _H), lambda i: (i, 0)),
    pl.BlockSpec((_NB, _H), lambda i: (i, 0)),
    pl.BlockSpec((_NB, _H), lambda i: (i, 0)),
    pl.BlockSpec((_NB, _H), lambda i: (i, 0)),
]
_W_SPEC = pl.BlockSpec((_H, 4 * _H), lambda i: (0, 0))
_B_SPEC = pl.BlockSpec((1, 4 * _H), lambda i: (0, 0))
_H_SPEC = pl.BlockSpec((_NB, _H), lambda i: (i, 0))


def _node_tf(h, wcat, bcat):
    grid = -(-_N // _NB)
    return pl.pallas_call(
        _node_tf_body,
        grid=(grid,),
        in_specs=[_H_SPEC, _W_SPEC, _B_SPEC],
        out_specs=_NODE_OUT_SPECS,
        out_shape=_NODE_OUT_SHAPES,
    )(h, wcat, bcat)


def _node_tf0(x, wx, bx, wcat, bcat):
    grid = -(-_N // _NB)
    return pl.pallas_call(
        _node_tf0_body,
        grid=(grid,),
        in_specs=[
            pl.BlockSpec((_NB, 2), lambda i: (i, 0)),
            pl.BlockSpec((2, _H), lambda i: (0, 0)),
            pl.BlockSpec((1, _H), lambda i: (0, 0)),
            _W_SPEC, _B_SPEC,
        ],
        out_specs=[_H_SPEC] + _NODE_OUT_SPECS,
        out_shape=[jax.ShapeDtypeStruct((_N, _H), jnp.float32)]
        + _NODE_OUT_SHAPES,
    )(x, wx, bx, wcat, bcat)


def _ln_relu(v, g, b):
    m = jnp.mean(v, axis=-1, keepdims=True)
    d = v - m
    s = jnp.mean(d * d, axis=-1, keepdims=True)
    ln = d * lax.rsqrt(s + 1e-5) * g + b
    return jnp.maximum(ln, 0.0)


def _node_up_body(nd0_ref, nd1_ref, ux_ref, h_ref, g_ref, b_ref, ho_ref):
    nd0 = nd0_ref[...]
    nd1 = nd1_ref[...]
    num = jnp.concatenate([nd0[:, :_HH], nd1[:, :_HH]], axis=1)
    den = jnp.concatenate([nd0[:, _HH:], nd1[:, _HH:]], axis=1)
    v = ux_ref[...] + num / (den + 1e-6)
    ho_ref[...] = h_ref[...] + _ln_relu(v, g_ref[...], b_ref[...])


def _node_up(nd0, nd1, ux, h, g, b):
    grid = -(-_N // _NB)
    return pl.pallas_call(
        _node_up_body,
        grid=(grid,),
        in_specs=[_H_SPEC, _H_SPEC, _H_SPEC, _H_SPEC,
                  pl.BlockSpec((1, _H), lambda i: (0, 0)),
                  pl.BlockSpec((1, _H), lambda i: (0, 0))],
        out_specs=_H_SPEC,
        out_shape=jax.ShapeDtypeStruct((_N, _H), jnp.float32),
    )(nd0, nd1, ux, h, g, b)


def _edge_body(he_ref, p_ref, a_ref, ba_ref, g_ref, b_ref, heo_ref):
    he = jnp.concatenate([he_ref[0], he_ref[1]], axis=1)
    p = jnp.concatenate([p_ref[0], p_ref[1]], axis=1)
    ein = (jnp.dot(he, a_ref[...], preferred_element_type=jnp.float32)
           + ba_ref[...] + p)
    heo = he + _ln_relu(ein, g_ref[...], b_ref[...])
    heo_ref[0, :, :] = heo[:, :_HH]
    heo_ref[1, :, :] = heo[:, _HH:]


_HE_SPEC = pl.BlockSpec((2, _EB, _HH), lambda i: (0, i, 0))
_HB_SPEC = pl.BlockSpec((_H, _H), lambda i: (0, 0))
_H1_SPEC = pl.BlockSpec((1, _H), lambda i: (0, 0))


def _tc_edge(he, p, a, ba, g, b):
    grid = _E // _EB
    return pl.pallas_call(
        _edge_body,
        grid=(grid,),
        in_specs=[_HE_SPEC, _HE_SPEC, _HB_SPEC, _H1_SPEC, _H1_SPEC, _H1_SPEC],
        out_specs=_HE_SPEC,
        out_shape=jax.ShapeDtypeStruct((2, _E, _HH), jnp.float32),
    )(he, p, a, ba, g, b)


def _edge_final_body(he_ref, p_ref, a_ref, ba_ref, g_ref, b_ref, wo_ref,
                     bo_ref, wl_ref, bl_ref, out_ref):
    he = jnp.concatenate([he_ref[0], he_ref[1]], axis=1)
    p = jnp.concatenate([p_ref[0], p_ref[1]], axis=1)
    ein = (jnp.dot(he, a_ref[...], preferred_element_type=jnp.float32)
           + ba_ref[...] + p)
    heo = he + _ln_relu(ein, g_ref[...], b_ref[...])
    z = jnp.maximum(
        jnp.dot(heo, wo_ref[...], preferred_element_type=jnp.float32)
        + bo_ref[...], 0.0)
    logit = jnp.sum(z * wl_ref[...], axis=1, keepdims=True) + bl_ref[...]
    out_ref[...] = jax.nn.sigmoid(logit)


def _tc_edge_final(he, p, a, ba, g, b, wo, bo, wl_row, bl):
    grid = _E // _EB
    return pl.pallas_call(
        _edge_final_body,
        grid=(grid,),
        in_specs=[_HE_SPEC, _HE_SPEC, _HB_SPEC, _H1_SPEC, _H1_SPEC, _H1_SPEC,
                  _HB_SPEC, _H1_SPEC, _H1_SPEC,
                  pl.BlockSpec((1, 1), lambda i: (0, 0))],
        out_specs=pl.BlockSpec((_EB, 1), lambda i: (i, 0)),
        out_shape=jax.ShapeDtypeStruct((_E, 1), jnp.float32),
    )(he, p, a, ba, g, b, wo, bo, wl_row, bl)


def _sc_body(he_hbm, tsrc_hbm, tdst_hbm, src_hbm, dst_hbm, nd_hbm, p_hbm,
             heb, sib, dib, tsb, tdb, vb, pb, zb, accum, sem1, sem2):
    c = lax.axis_index("c")
    s = lax.axis_index("s")
    cn = c * _N

    def _zrow(r, carry):
        for k in range(_H // 16):
            zb[r, pl.ds(k * 16, 16)] = jnp.zeros((16,), jnp.float32)
        return carry
    lax.fori_loop(0, _RC, _zrow, 0)
    for t in range(_NROW // _RC):
        pltpu.sync_copy(zb, accum.at[pl.ds(s * _NROW + t * _RC, _RC)])
    plsc.subcore_barrier()

    def _chunk(i, carry):
        cid = s + _NS * i

        @pl.when(cid < _NCHUNK)
        def _():
            base = cid * _CE
            pltpu.sync_copy(he_hbm.at[c, pl.ds(base, _CE)], heb)
            pltpu.sync_copy(src_hbm.at[pl.ds(base, _CE)], sib)
            pltpu.sync_copy(dst_hbm.at[pl.ds(base, _CE)], dib)

            def _addcn(r, carry2):
                sib[pl.ds(r * 16, 16)] = sib[pl.ds(r * 16, 16)] + cn
                return carry2
            lax.fori_loop(0, _CE // 16, _addcn, 0)

            cp1 = pltpu.async_copy(tsrc_hbm.at[sib], tsb, sem1)
            cp2 = pltpu.async_copy(tdst_hbm.at[dib], tdb, sem2)
            cp1.wait()
            cp2.wait()

            def _row(r, carry2):
                for k in range(_HH // 16):
                    x = heb[r, pl.ds(k * 16, 16)]
                    g = 1.0 / (1.0 + jnp.exp(-x))
                    vx = tsb[r, pl.ds(k * 16, 16)]
                    hb = tsb[r, pl.ds(_HH + k * 16, 16)]
                    hc = tdb[r, pl.ds(c * _HH + k * 16, 16)]
                    vb[r, pl.ds(k * 16, 16)] = g * vx
                    vb[r, pl.ds(_HH + k * 16, 16)] = g
                    pb[r, pl.ds(k * 16, 16)] = hb + hc
                return carry2
            lax.fori_loop(0, _CE, _row, 0)

            pltpu.sync_copy(vb, accum.at[dib], add=True)
            pltpu.sync_copy(pb, p_hbm.at[c, pl.ds(base, _CE)])
        return carry
    lax.fori_loop(0, _ITERS, _chunk, 0)

    plsc.subcore_barrier()
    for t in range(_NROW // _RC):
        r0 = s * _NROW + t * _RC
        pltpu.sync_copy(accum.at[pl.ds(r0, _RC)], nd_hbm.at[c, pl.ds(r0, _RC)])


@functools.lru_cache(maxsize=None)
def _build_sc_edge_pass():
    return pl.kernel(
        _sc_body,
        out_type=[
            jax.ShapeDtypeStruct((2, _N, _H), jnp.float32),
            jax.ShapeDtypeStruct((2, _E, _HH), jnp.float32),
        ],
        mesh=plsc.VectorSubcoreMesh(core_axis_name="c", subcore_axis_name="s"),
        compiler_params=pltpu.CompilerParams(use_tc_tiling_on_sc=False),
        scratch_types=[
            pltpu.VMEM((_CE, _HH), jnp.float32),
            pltpu.VMEM((_CE,), jnp.int32),
            pltpu.VMEM((_CE,), jnp.int32),
            pltpu.VMEM((_CE, _H), jnp.float32),
            pltpu.VMEM((_CE, _H), jnp.float32),
            pltpu.VMEM((_CE, _H), jnp.float32),
            pltpu.VMEM((_CE, _HH), jnp.float32),
            pltpu.VMEM((_RC, _H), jnp.float32),
            pltpu.VMEM_SHARED((_N, _H), jnp.float32),
            pltpu.SemaphoreType.DMA,
            pltpu.SemaphoreType.DMA,
        ],
    )


def _sc_edge_pass(he, tsrc, tdst, src, dst):
    return _build_sc_edge_pass()(he, tsrc, tdst, src, dst)


def kernel(x, e, edge_index, We_x, be_x, We_e, be_e, U, bU, V, bV, A, bA,
           B, bB, C, bC, gx, bx, ge, bep, Wo, bo, Wl, bl):
    src = edge_index[0]
    dst = edge_index[1]

    he = _embed_edge(e.reshape(_E, 1), We_e, be_e.reshape(1, _H))

    h = None
    out = None
    for l in range(3):
        wcat = jnp.concatenate([U[l], V[l], B[l], C[l]], axis=1)
        bcat = jnp.concatenate([bU[l], bV[l], bB[l], bC[l]]).reshape(1, 4 * _H)
        if l == 0:
            h, ux, tsa, tsb, tdst = _node_tf0(
                x, We_x, be_x.reshape(1, _H), wcat, bcat)
        else:
            ux, tsa, tsb, tdst = _node_tf(h, wcat, bcat)
        tsrc = jnp.concatenate([tsa, tsb], axis=0)

        nd, p = _sc_edge_pass(he, tsrc, tdst, src, dst)

        h = _node_up(nd[0], nd[1], ux, h, gx[l].reshape(1, _H),
                     bx[l].reshape(1, _H))
        if l < 2:
            he = _tc_edge(he, p, A[l], bA[l].reshape(1, _H),
                          ge[l].reshape(1, _H), bep[l].reshape(1, _H))
        else:
            out = _tc_edge_final(
                he, p, A[l], bA[l].reshape(1, _H), ge[l].reshape(1, _H),
                bep[l].reshape(1, _H), Wo, bo.reshape(1, _H),
                Wl.reshape(1, _H), bl.reshape(1, 1))
    return out.reshape(_E)
